# Initial kernel scaffold; baseline (speedup 1.0000x reference)
import jax
import jax.numpy as jnp
from jax.experimental import pallas as pl


def kernel(feature, edge_index, labels, W_emb, b_emb, Wg, bg, gam, bet, Wmg0, bmg0, Wmg1, bmg1, Wmg2, bmg2, Wmn0, bmn0, Wmn1, bmn1, Wmn2, bmn2):
    raise NotImplementedError("write your pallas kernel here")


# speedup vs baseline: 2.7995x; 2.7995x over previous
import functools

import jax
import jax.numpy as jnp
from jax import lax
from jax.experimental import pallas as pl
from jax.experimental.pallas import tpu as pltpu
from jax.experimental.pallas import tpu_sc as plsc

N = 10000
E = 320000
G = 10
NPG = N // G
IN_DIM = 128
H = 128
NC_CLS = 10
NL = 3

NCORES = 2
NSUB = 16
NW = NCORES * NSUB
EPS = E // NSUB
CH = 80
NCHUNK = EPS // CH
HR = 5120
ACCR = HR + 8
RPC = HR // NSUB
ZB = 64
DW = 128


def _sc_degrees_body(srcm_hbm, dstm_hbm, ones_hbm, zrows_hbm, out_hbm,
                     srcm_v, dstm_v, ones_v, buf_v, acc_sh):
    c = lax.axis_index("c")
    s = lax.axis_index("s")
    wid = c * NSUB + s
    pltpu.sync_copy(srcm_hbm.at[wid], srcm_v)
    pltpu.sync_copy(dstm_hbm.at[wid], dstm_v)
    pltpu.sync_copy(ones_hbm, ones_v)
    r0 = s * RPC
    for p, idx_v in ((0, srcm_v), (1, dstm_v)):
        pltpu.sync_copy(zrows_hbm, buf_v)
        for k in range(RPC // ZB):
            pltpu.sync_copy(buf_v, acc_sh.at[pl.ds(r0 + k * ZB, ZB)])
        plsc.subcore_barrier()

        def body(j, carry):
            pltpu.sync_copy(ones_v, acc_sh.at[idx_v.at[j]], add=True)
            return carry

        lax.fori_loop(0, NCHUNK, body, 0)
        plsc.subcore_barrier()
        for k in range(RPC // ZB):
            pltpu.sync_copy(acc_sh.at[pl.ds(r0 + k * ZB, ZB)], buf_v)
            pltpu.sync_copy(buf_v, out_hbm.at[c, p, pl.ds(r0 + k * ZB, ZB)])
        plsc.subcore_barrier()


def _sc_segsum_body(src_hbm, dstm_hbm, h_hbm, zrows_hbm, out_hbm,
                    src_v, dst_v, rows_v, buf_v, acc_sh, sem):
    c = lax.axis_index("c")
    s = lax.axis_index("s")
    wid = c * NSUB + s
    pltpu.sync_copy(src_hbm.at[wid], src_v)
    pltpu.sync_copy(dstm_hbm.at[wid], dst_v)
    pltpu.sync_copy(zrows_hbm, buf_v)
    r0 = s * RPC
    for k in range(RPC // ZB):
        pltpu.sync_copy(buf_v, acc_sh.at[pl.ds(r0 + k * ZB, ZB)])
    plsc.subcore_barrier()

    def body(j, carry):
        pltpu.async_copy(h_hbm.at[src_v.at[j]], rows_v, sem).wait()
        pltpu.sync_copy(rows_v, acc_sh.at[dst_v.at[j]], add=True)
        return carry

    lax.fori_loop(0, NCHUNK, body, 0)
    plsc.subcore_barrier()
    for k in range(RPC // ZB):
        pltpu.sync_copy(acc_sh.at[pl.ds(r0 + k * ZB, ZB)], buf_v)
        pltpu.sync_copy(buf_v, out_hbm.at[c, pl.ds(r0 + k * ZB, ZB)])


@functools.cache
def _sc_kernels():
    mesh = plsc.VectorSubcoreMesh(
        core_axis_name="c", subcore_axis_name="s",
        num_cores=NCORES, num_subcores=NSUB)
    sc_degrees = pl.kernel(
        _sc_degrees_body,
        out_type=jax.ShapeDtypeStruct((NCORES, 2, HR, DW), jnp.float32),
        mesh=mesh,
        scratch_types=[
            pltpu.VMEM((NCHUNK, CH), jnp.int32),
            pltpu.VMEM((NCHUNK, CH), jnp.int32),
            pltpu.VMEM((CH, DW), jnp.float32),
            pltpu.VMEM((ZB, DW), jnp.float32),
            pltpu.VMEM_SHARED((ACCR, DW), jnp.float32),
        ],
    )
    sc_segsum = pl.kernel(
        _sc_segsum_body,
        out_type=jax.ShapeDtypeStruct((NCORES, HR, H), jnp.float32),
        mesh=mesh,
        scratch_types=[
            pltpu.VMEM((NCHUNK, CH), jnp.int32),
            pltpu.VMEM((NCHUNK, CH), jnp.int32),
            pltpu.VMEM((CH, H), jnp.float32),
            pltpu.VMEM((ZB, H), jnp.float32),
            pltpu.VMEM_SHARED((ACCR, H), jnp.float32),
            pltpu.SemaphoreType.DMA,
        ],
    )
    return sc_degrees, sc_segsum


def _degrees_from(degp):
    out_deg = jnp.concatenate(
        [degp[0, 0, :, 0:1], degp[1, 0, 0:N - HR, 0:1]], axis=0)
    in_deg = jnp.concatenate(
        [degp[0, 1, :, 0:1], degp[1, 1, 0:N - HR, 0:1]], axis=0)
    return out_deg, in_deg


def _tc_prep_body(feat, wemb, bemb, wg0, bg0, degp,
                  x0_o, hpre_o, ns_o, nd_o):
    x0 = jnp.dot(feat[...], wemb[...], preferred_element_type=jnp.float32)
    x0 = x0 + bemb[...]
    out_deg, in_deg = _degrees_from(degp)
    ns = lax.rsqrt(jnp.maximum(out_deg, 1.0))
    nd = lax.rsqrt(jnp.maximum(in_deg, 1.0))
    x0_o[...] = x0
    hpre_o[...] = (jnp.dot(x0, wg0[...], preferred_element_type=jnp.float32)
                   + bg0[...]) * ns
    ns_o[...] = ns
    nd_o[...] = nd


_tc_prep = pl.pallas_call(
    _tc_prep_body,
    out_shape=[
        jax.ShapeDtypeStruct((N, H), jnp.float32),
        jax.ShapeDtypeStruct((N, H), jnp.float32),
        jax.ShapeDtypeStruct((N, 1), jnp.float32),
        jax.ShapeDtypeStruct((N, 1), jnp.float32),
    ],
)


def _bn_relu_residual(p, x, nd, gam, bet):
    agg = jnp.concatenate([p[0], p[1, 0:N - HR]], axis=0)
    h = agg * nd[...]
    mu = jnp.mean(h, axis=0, keepdims=True)
    hc = h - mu
    var = jnp.mean(hc * hc, axis=0, keepdims=True)
    hn = gam[...] * hc * lax.rsqrt(var + 1e-5) + bet[...]
    return x[...] + jnp.maximum(hn, 0.0)


def _tc_layer_body(p, x, nd, gam, bet, wgn, bgn, ns, xo, hpo):
    xn = _bn_relu_residual(p, x, nd, gam, bet)
    xo[...] = xn
    hpo[...] = (jnp.dot(xn, wgn[...], preferred_element_type=jnp.float32)
                + bgn[...]) * ns[...]


_tc_layer = pl.pallas_call(
    _tc_layer_body,
    out_shape=[
        jax.ShapeDtypeStruct((N, H), jnp.float32),
        jax.ShapeDtypeStruct((N, H), jnp.float32),
    ],
)


def _tc_final_body(p, x, nd, gam, bet,
                   wmg0, bmg0, wmg1, bmg1, wmg2, bmg2,
                   wmn0, bmn0, wmn1, bmn1, wmn2, bmn2,
                   xo, xgo, go):
    x3 = _bn_relu_residual(p, x, nd, gam, bet)
    xo[...] = x3
    xr = x3.reshape(G, NPG, H)
    avg = jnp.mean(xr, axis=1)
    mx = jnp.max(xr, axis=1)
    hg = jnp.concatenate([avg, mx], axis=-1)

    g = jnp.dot(hg, wmg0[...], preferred_element_type=jnp.float32) + bmg0[...]
    g = jnp.maximum(g, 0.0)
    g = jnp.dot(g, wmg1[...], preferred_element_type=jnp.float32) + bmg1[...]
    g = jnp.maximum(g, 0.0)
    go[...] = jnp.dot(g, wmg2[...], preferred_element_type=jnp.float32) + bmg2[...]

    hgr = jnp.broadcast_to(hg[:, None, :], (G, NPG, 2 * H)).reshape(N, 2 * H)
    xin = jnp.concatenate([hgr, x3], axis=1)
    y = jnp.dot(xin, wmn0[...], preferred_element_type=jnp.float32) + bmn0[...]
    y = jnp.maximum(y, 0.0)
    y = jnp.dot(y, wmn1[...], preferred_element_type=jnp.float32) + bmn1[...]
    y = jnp.maximum(y, 0.0)
    xgo[...] = jnp.dot(y, wmn2[...], preferred_element_type=jnp.float32) + bmn2[...]


_tc_final = pl.pallas_call(
    _tc_final_body,
    out_shape=[
        jax.ShapeDtypeStruct((N, H), jnp.float32),
        jax.ShapeDtypeStruct((N, NC_CLS), jnp.float32),
        jax.ShapeDtypeStruct((G, NC_CLS), jnp.float32),
    ],
)


def kernel(feature, edge_index, labels, W_emb, b_emb, Wg, bg, gam, bet,
           Wmg0, bmg0, Wmg1, bmg1, Wmg2, bmg2,
           Wmn0, bmn0, Wmn1, bmn1, Wmn2, bmn2):
    src = edge_index[0]
    dst = edge_index[1]
    srcr = src.reshape(NSUB, NCHUNK, CH)
    src_w = jnp.broadcast_to(
        srcr[None], (NCORES, NSUB, NCHUNK, CH)).reshape(NW, NCHUNK, CH)

    bases = (jnp.arange(NCORES, dtype=jnp.int32) * HR).reshape(NCORES, 1, 1, 1)

    def core_masked(idx):
        loc = idx.reshape(1, NSUB, NCHUNK, CH) - bases
        ok = (loc >= 0) & (loc < HR)
        return jnp.where(ok, loc, HR).astype(jnp.int32).reshape(
            NW, NCHUNK, CH)

    srcm_w = core_masked(src)
    dstm_w = core_masked(dst)

    ones = jnp.ones((CH, DW), jnp.float32)
    zdeg = jnp.zeros((ZB, DW), jnp.float32)
    zrows = jnp.zeros((ZB, H), jnp.float32)

    sc_degrees, sc_segsum = _sc_kernels()
    degp = sc_degrees(srcm_w, dstm_w, ones, zdeg)
    x, hpre, ns, nd = _tc_prep(feature, W_emb, b_emb, Wg[0], bg[0], degp)

    x3 = x_g = g = None
    for l in range(NL):
        p = sc_segsum(src_w, dstm_w, hpre, zrows)
        if l < NL - 1:
            x, hpre = _tc_layer(p, x, nd, gam[l], bet[l],
                                Wg[l + 1], bg[l + 1], ns)
        else:
            x3, x_g, g = _tc_final(p, x, nd, gam[l], bet[l],
                                   Wmg0, bmg0, Wmg1, bmg1, Wmg2, bmg2,
                                   Wmn0, bmn0, Wmn1, bmn1, Wmn2, bmn2)

    node_label = jnp.repeat(labels, NPG, axis=0)
    return x3, x_g, g, node_label


# speedup vs baseline: 3.1291x; 1.1178x over previous
import functools

import jax
import jax.numpy as jnp
from jax import lax
from jax.experimental import pallas as pl
from jax.experimental.pallas import tpu as pltpu
from jax.experimental.pallas import tpu_sc as plsc

N = 10000
E = 320000
G = 10
NPG = N // G
IN_DIM = 128
H = 128
NC_CLS = 10
NL = 3

NCORES = 2
NSUB = 16
NW = NCORES * NSUB
EPS = E // NSUB
CH = 80
NCHUNK = EPS // CH
HR = 5120
ACCR = HR + 80
RPC = HR // NSUB
ZB = 64
DW = 128


def _sc_degrees_body(srcm_hbm, dstm_hbm, ones_hbm, zrows_hbm, out_hbm,
                     srcm_v, dstm_v, ones_v, buf_v, acc_sh):
    c = lax.axis_index("c")
    s = lax.axis_index("s")
    wid = c * NSUB + s
    pltpu.sync_copy(srcm_hbm.at[wid], srcm_v)
    pltpu.sync_copy(dstm_hbm.at[wid], dstm_v)
    pltpu.sync_copy(ones_hbm, ones_v)
    r0 = s * RPC
    for p, idx_v in ((0, srcm_v), (1, dstm_v)):
        pltpu.sync_copy(zrows_hbm, buf_v)
        for k in range(RPC // ZB):
            pltpu.sync_copy(buf_v, acc_sh.at[pl.ds(r0 + k * ZB, ZB)])
        plsc.subcore_barrier()

        def body(j, carry):
            pltpu.sync_copy(ones_v, acc_sh.at[idx_v.at[j]], add=True)
            return carry

        lax.fori_loop(0, NCHUNK, body, 0)
        plsc.subcore_barrier()
        for k in range(RPC // ZB):
            pltpu.sync_copy(acc_sh.at[pl.ds(r0 + k * ZB, ZB)], buf_v)
            pltpu.sync_copy(buf_v, out_hbm.at[c, p, pl.ds(r0 + k * ZB, ZB)])
        plsc.subcore_barrier()


def _sc_segsum_body(src_hbm, dstm_hbm, h_hbm, zrows_hbm, out_hbm,
                    src_v, dst_v, rows_v, buf_v, acc_sh, sem):
    c = lax.axis_index("c")
    s = lax.axis_index("s")
    wid = c * NSUB + s
    pltpu.sync_copy(src_hbm.at[wid], src_v)
    pltpu.sync_copy(dstm_hbm.at[wid], dst_v)
    pltpu.sync_copy(zrows_hbm, buf_v)
    r0 = s * RPC
    for k in range(RPC // ZB):
        pltpu.sync_copy(buf_v, acc_sh.at[pl.ds(r0 + k * ZB, ZB)])
    plsc.subcore_barrier()

    def body(j, carry):
        pltpu.async_copy(h_hbm.at[src_v.at[j]], rows_v, sem).wait()
        pltpu.sync_copy(rows_v, acc_sh.at[dst_v.at[j]], add=True)
        return carry

    lax.fori_loop(0, NCHUNK, body, 0)
    plsc.subcore_barrier()
    for k in range(RPC // ZB):
        pltpu.sync_copy(acc_sh.at[pl.ds(r0 + k * ZB, ZB)], buf_v)
        pltpu.sync_copy(buf_v, out_hbm.at[c, pl.ds(r0 + k * ZB, ZB)])


@functools.cache
def _sc_kernels():
    mesh = plsc.VectorSubcoreMesh(
        core_axis_name="c", subcore_axis_name="s",
        num_cores=NCORES, num_subcores=NSUB)
    sc_degrees = pl.kernel(
        _sc_degrees_body,
        out_type=jax.ShapeDtypeStruct((NCORES, 2, HR, DW), jnp.float32),
        mesh=mesh,
        scratch_types=[
            pltpu.VMEM((NCHUNK, CH), jnp.int32),
            pltpu.VMEM((NCHUNK, CH), jnp.int32),
            pltpu.VMEM((CH, DW), jnp.float32),
            pltpu.VMEM((ZB, DW), jnp.float32),
            pltpu.VMEM_SHARED((ACCR, DW), jnp.float32),
        ],
    )
    sc_segsum = pl.kernel(
        _sc_segsum_body,
        out_type=jax.ShapeDtypeStruct((NCORES, HR, H), jnp.float32),
        mesh=mesh,
        scratch_types=[
            pltpu.VMEM((NCHUNK, CH), jnp.int32),
            pltpu.VMEM((NCHUNK, CH), jnp.int32),
            pltpu.VMEM((CH, H), jnp.float32),
            pltpu.VMEM((ZB, H), jnp.float32),
            pltpu.VMEM_SHARED((ACCR, H), jnp.float32),
            pltpu.SemaphoreType.DMA,
        ],
    )
    return sc_degrees, sc_segsum


def _degrees_from(degp):
    out_deg = jnp.concatenate(
        [degp[0, 0, :, 0:1], degp[1, 0, 0:N - HR, 0:1]], axis=0)
    in_deg = jnp.concatenate(
        [degp[0, 1, :, 0:1], degp[1, 1, 0:N - HR, 0:1]], axis=0)
    return out_deg, in_deg


def _tc_prep_body(feat, wemb, bemb, wg0, bg0, degp,
                  x0_o, hpre_o, ns_o, nd_o):
    x0 = jnp.dot(feat[...], wemb[...], preferred_element_type=jnp.float32)
    x0 = x0 + bemb[...]
    out_deg, in_deg = _degrees_from(degp)
    ns = lax.rsqrt(jnp.maximum(out_deg, 1.0))
    nd = lax.rsqrt(jnp.maximum(in_deg, 1.0))
    x0_o[...] = x0
    hpre_o[...] = (jnp.dot(x0, wg0[...], preferred_element_type=jnp.float32)
                   + bg0[...]) * ns
    ns_o[...] = ns
    nd_o[...] = nd


_tc_prep = pl.pallas_call(
    _tc_prep_body,
    out_shape=[
        jax.ShapeDtypeStruct((N, H), jnp.float32),
        jax.ShapeDtypeStruct((N, H), jnp.float32),
        jax.ShapeDtypeStruct((N, 1), jnp.float32),
        jax.ShapeDtypeStruct((N, 1), jnp.float32),
    ],
)


def _bn_relu_residual(p, x, nd, gam, bet):
    agg = jnp.concatenate([p[0], p[1, 0:N - HR]], axis=0)
    h = agg * nd[...]
    mu = jnp.mean(h, axis=0, keepdims=True)
    hc = h - mu
    var = jnp.mean(hc * hc, axis=0, keepdims=True)
    hn = gam[...] * hc * lax.rsqrt(var + 1e-5) + bet[...]
    return x[...] + jnp.maximum(hn, 0.0)


def _tc_layer_body(p, x, nd, gam, bet, wgn, bgn, ns, xo, hpo):
    xn = _bn_relu_residual(p, x, nd, gam, bet)
    xo[...] = xn
    hpo[...] = (jnp.dot(xn, wgn[...], preferred_element_type=jnp.float32)
                + bgn[...]) * ns[...]


_tc_layer = pl.pallas_call(
    _tc_layer_body,
    out_shape=[
        jax.ShapeDtypeStruct((N, H), jnp.float32),
        jax.ShapeDtypeStruct((N, H), jnp.float32),
    ],
)


def _tc_final_body(p, x, nd, gam, bet,
                   wmg0, bmg0, wmg1, bmg1, wmg2, bmg2,
                   wmn0, bmn0, wmn1, bmn1, wmn2, bmn2,
                   xo, xgo, go):
    x3 = _bn_relu_residual(p, x, nd, gam, bet)
    xo[...] = x3
    xr = x3.reshape(G, NPG, H)
    avg = jnp.mean(xr, axis=1)
    mx = jnp.max(xr, axis=1)
    hg = jnp.concatenate([avg, mx], axis=-1)

    g = jnp.dot(hg, wmg0[...], preferred_element_type=jnp.float32) + bmg0[...]
    g = jnp.maximum(g, 0.0)
    g = jnp.dot(g, wmg1[...], preferred_element_type=jnp.float32) + bmg1[...]
    g = jnp.maximum(g, 0.0)
    go[...] = jnp.dot(g, wmg2[...], preferred_element_type=jnp.float32) + bmg2[...]

    hgr = jnp.broadcast_to(hg[:, None, :], (G, NPG, 2 * H)).reshape(N, 2 * H)
    xin = jnp.concatenate([hgr, x3], axis=1)
    y = jnp.dot(xin, wmn0[...], preferred_element_type=jnp.float32) + bmn0[...]
    y = jnp.maximum(y, 0.0)
    y = jnp.dot(y, wmn1[...], preferred_element_type=jnp.float32) + bmn1[...]
    y = jnp.maximum(y, 0.0)
    xgo[...] = jnp.dot(y, wmn2[...], preferred_element_type=jnp.float32) + bmn2[...]


_tc_final = pl.pallas_call(
    _tc_final_body,
    out_shape=[
        jax.ShapeDtypeStruct((N, H), jnp.float32),
        jax.ShapeDtypeStruct((N, NC_CLS), jnp.float32),
        jax.ShapeDtypeStruct((G, NC_CLS), jnp.float32),
    ],
)


def kernel(feature, edge_index, labels, W_emb, b_emb, Wg, bg, gam, bet,
           Wmg0, bmg0, Wmg1, bmg1, Wmg2, bmg2,
           Wmn0, bmn0, Wmn1, bmn1, Wmn2, bmn2):
    src = edge_index[0]
    dst = edge_index[1]
    srcr = src.reshape(NSUB, NCHUNK, CH)
    src_w = jnp.broadcast_to(
        srcr[None], (NCORES, NSUB, NCHUNK, CH)).reshape(NW, NCHUNK, CH)

    bases = (jnp.arange(NCORES, dtype=jnp.int32) * HR).reshape(NCORES, 1, 1, 1)

    dummy = HR + jnp.arange(CH, dtype=jnp.int32)

    def core_masked(idx):
        loc = idx.reshape(1, NSUB, NCHUNK, CH) - bases
        ok = (loc >= 0) & (loc < HR)
        return jnp.where(ok, loc, dummy).astype(jnp.int32).reshape(
            NW, NCHUNK, CH)

    srcm_w = core_masked(src)
    dstm_w = core_masked(dst)

    ones = jnp.ones((CH, DW), jnp.float32)
    zdeg = jnp.zeros((ZB, DW), jnp.float32)
    zrows = jnp.zeros((ZB, H), jnp.float32)

    sc_degrees, sc_segsum = _sc_kernels()
    degp = sc_degrees(srcm_w, dstm_w, ones, zdeg)
    x, hpre, ns, nd = _tc_prep(feature, W_emb, b_emb, Wg[0], bg[0], degp)

    x3 = x_g = g = None
    for l in range(NL):
        p = sc_segsum(src_w, dstm_w, hpre, zrows)
        if l < NL - 1:
            x, hpre = _tc_layer(p, x, nd, gam[l], bet[l],
                                Wg[l + 1], bg[l + 1], ns)
        else:
            x3, x_g, g = _tc_final(p, x, nd, gam[l], bet[l],
                                   Wmg0, bmg0, Wmg1, bmg1, Wmg2, bmg2,
                                   Wmn0, bmn0, Wmn1, bmn1, Wmn2, bmn2)

    node_label = jnp.repeat(labels, NPG, axis=0)
    return x3, x_g, g, node_label


# speedup vs baseline: 4.7735x; 1.5255x over previous
import functools

import jax
import jax.numpy as jnp
from jax import lax
from jax.experimental import pallas as pl
from jax.experimental.pallas import tpu as pltpu
from jax.experimental.pallas import tpu_sc as plsc

N = 10000
E = 320000
G = 10
NPG = N // G
IN_DIM = 128
H = 128
NC_CLS = 10
NL = 3

NCORES = 2
NSUB = 16
NW = NCORES * NSUB
EPS = E // NSUB
CH = 80
NCHUNK = EPS // CH
HR = 5120
ACCR = HR + 80
RPC = HR // NSUB
ZB = 64
NBUF = 2
DW = 128


def _sc_degrees_body(srcm_hbm, dstm_hbm, ones_hbm, zrows_hbm, out_hbm,
                     srcm_v, dstm_v, ones_v, buf_v, acc_sh):
    c = lax.axis_index("c")
    s = lax.axis_index("s")
    wid = c * NSUB + s
    pltpu.sync_copy(srcm_hbm.at[wid], srcm_v)
    pltpu.sync_copy(dstm_hbm.at[wid], dstm_v)
    pltpu.sync_copy(ones_hbm, ones_v)
    r0 = s * RPC
    for p, idx_v in ((0, srcm_v), (1, dstm_v)):
        pltpu.sync_copy(zrows_hbm, buf_v)
        for k in range(RPC // ZB):
            pltpu.sync_copy(buf_v, acc_sh.at[pl.ds(r0 + k * ZB, ZB)])
        plsc.subcore_barrier()

        def body(j, carry):
            pltpu.sync_copy(ones_v, acc_sh.at[idx_v.at[j]], add=True)
            return carry

        lax.fori_loop(0, NCHUNK, body, 0)
        plsc.subcore_barrier()
        for k in range(RPC // ZB):
            pltpu.sync_copy(acc_sh.at[pl.ds(r0 + k * ZB, ZB)], buf_v)
            pltpu.sync_copy(buf_v, out_hbm.at[c, p, pl.ds(r0 + k * ZB, ZB)])
        plsc.subcore_barrier()


def _sc_segsum_body(src_hbm, dstm_hbm, h_hbm, zrows_hbm, out_hbm,
                    src_v, dst_v, rows_v, acc_sh, sem0, sem1):
    c = lax.axis_index("c")
    s = lax.axis_index("s")
    wid = c * NSUB + s
    sems = (sem0, sem1)
    pltpu.sync_copy(src_hbm.at[wid], src_v)
    pltpu.sync_copy(dstm_hbm.at[wid], dst_v)
    pltpu.sync_copy(zrows_hbm, rows_v.at[0])
    r0 = s * RPC
    for k in range(RPC // CH):
        pltpu.sync_copy(rows_v.at[0], acc_sh.at[pl.ds(r0 + k * CH, CH)])
    plsc.subcore_barrier()

    for b in range(NBUF):
        pltpu.async_copy(h_hbm.at[src_v.at[b]], rows_v.at[b], sems[b])

    def body(g, carry):
        for b in range(NBUF):
            j = g * NBUF + b
            pltpu.make_async_copy(
                h_hbm.at[src_v.at[j]], rows_v.at[b], sems[b]).wait()
            pltpu.sync_copy(rows_v.at[b], acc_sh.at[dst_v.at[j]], add=True)
            jn = jnp.minimum(j + NBUF, NCHUNK - 1)
            pltpu.async_copy(h_hbm.at[src_v.at[jn]], rows_v.at[b], sems[b])
        return carry

    lax.fori_loop(0, NCHUNK // NBUF, body, 0)
    for b in range(NBUF):
        pltpu.make_async_copy(
            h_hbm.at[src_v.at[NCHUNK - 1]], rows_v.at[b], sems[b]).wait()
    plsc.subcore_barrier()
    for k in range(RPC // CH):
        pltpu.sync_copy(acc_sh.at[pl.ds(r0 + k * CH, CH)], rows_v.at[0])
        pltpu.sync_copy(rows_v.at[0], out_hbm.at[c, pl.ds(r0 + k * CH, CH)])


@functools.cache
def _sc_kernels():
    mesh = plsc.VectorSubcoreMesh(
        core_axis_name="c", subcore_axis_name="s",
        num_cores=NCORES, num_subcores=NSUB)
    sc_degrees = pl.kernel(
        _sc_degrees_body,
        out_type=jax.ShapeDtypeStruct((NCORES, 2, HR, DW), jnp.float32),
        mesh=mesh,
        scratch_types=[
            pltpu.VMEM((NCHUNK, CH), jnp.int32),
            pltpu.VMEM((NCHUNK, CH), jnp.int32),
            pltpu.VMEM((CH, DW), jnp.float32),
            pltpu.VMEM((ZB, DW), jnp.float32),
            pltpu.VMEM_SHARED((ACCR, DW), jnp.float32),
        ],
    )
    sc_segsum = pl.kernel(
        _sc_segsum_body,
        out_type=jax.ShapeDtypeStruct((NCORES, HR, H), jnp.float32),
        mesh=mesh,
        scratch_types=[
            pltpu.VMEM((NCHUNK, CH), jnp.int32),
            pltpu.VMEM((NCHUNK, CH), jnp.int32),
            pltpu.VMEM((NBUF, CH, H), jnp.float32),
            pltpu.VMEM_SHARED((ACCR, H), jnp.float32),
            pltpu.SemaphoreType.DMA,
            pltpu.SemaphoreType.DMA,
        ],
    )
    return sc_degrees, sc_segsum


def _degrees_from(degp):
    out_deg = jnp.concatenate(
        [degp[0, 0, :, 0:1], degp[1, 0, 0:N - HR, 0:1]], axis=0)
    in_deg = jnp.concatenate(
        [degp[0, 1, :, 0:1], degp[1, 1, 0:N - HR, 0:1]], axis=0)
    return out_deg, in_deg


def _tc_prep_body(feat, wemb, bemb, wg0, bg0, degp,
                  x0_o, hpre_o, ns_o, nd_o):
    x0 = jnp.dot(feat[...], wemb[...], preferred_element_type=jnp.float32)
    x0 = x0 + bemb[...]
    out_deg, in_deg = _degrees_from(degp)
    ns = lax.rsqrt(jnp.maximum(out_deg, 1.0))
    nd = lax.rsqrt(jnp.maximum(in_deg, 1.0))
    x0_o[...] = x0
    hpre_o[...] = (jnp.dot(x0, wg0[...], preferred_element_type=jnp.float32)
                   + bg0[...]) * ns
    ns_o[...] = ns
    nd_o[...] = nd


_tc_prep = pl.pallas_call(
    _tc_prep_body,
    out_shape=[
        jax.ShapeDtypeStruct((N, H), jnp.float32),
        jax.ShapeDtypeStruct((N, H), jnp.float32),
        jax.ShapeDtypeStruct((N, 1), jnp.float32),
        jax.ShapeDtypeStruct((N, 1), jnp.float32),
    ],
)


def _bn_relu_residual(p, x, nd, gam, bet):
    agg = jnp.concatenate([p[0], p[1, 0:N - HR]], axis=0)
    h = agg * nd[...]
    mu = jnp.mean(h, axis=0, keepdims=True)
    hc = h - mu
    var = jnp.mean(hc * hc, axis=0, keepdims=True)
    hn = gam[...] * hc * lax.rsqrt(var + 1e-5) + bet[...]
    return x[...] + jnp.maximum(hn, 0.0)


def _tc_layer_body(p, x, nd, gam, bet, wgn, bgn, ns, xo, hpo):
    xn = _bn_relu_residual(p, x, nd, gam, bet)
    xo[...] = xn
    hpo[...] = (jnp.dot(xn, wgn[...], preferred_element_type=jnp.float32)
                + bgn[...]) * ns[...]


_tc_layer = pl.pallas_call(
    _tc_layer_body,
    out_shape=[
        jax.ShapeDtypeStruct((N, H), jnp.float32),
        jax.ShapeDtypeStruct((N, H), jnp.float32),
    ],
)


def _tc_final_body(p, x, nd, gam, bet,
                   wmg0, bmg0, wmg1, bmg1, wmg2, bmg2,
                   wmn0, bmn0, wmn1, bmn1, wmn2, bmn2,
                   xo, xgo, go):
    x3 = _bn_relu_residual(p, x, nd, gam, bet)
    xo[...] = x3
    xr = x3.reshape(G, NPG, H)
    avg = jnp.mean(xr, axis=1)
    mx = jnp.max(xr, axis=1)
    hg = jnp.concatenate([avg, mx], axis=-1)

    g = jnp.dot(hg, wmg0[...], preferred_element_type=jnp.float32) + bmg0[...]
    g = jnp.maximum(g, 0.0)
    g = jnp.dot(g, wmg1[...], preferred_element_type=jnp.float32) + bmg1[...]
    g = jnp.maximum(g, 0.0)
    go[...] = jnp.dot(g, wmg2[...], preferred_element_type=jnp.float32) + bmg2[...]

    hgr = jnp.broadcast_to(hg[:, None, :], (G, NPG, 2 * H)).reshape(N, 2 * H)
    xin = jnp.concatenate([hgr, x3], axis=1)
    y = jnp.dot(xin, wmn0[...], preferred_element_type=jnp.float32) + bmn0[...]
    y = jnp.maximum(y, 0.0)
    y = jnp.dot(y, wmn1[...], preferred_element_type=jnp.float32) + bmn1[...]
    y = jnp.maximum(y, 0.0)
    xgo[...] = jnp.dot(y, wmn2[...], preferred_element_type=jnp.float32) + bmn2[...]


_tc_final = pl.pallas_call(
    _tc_final_body,
    out_shape=[
        jax.ShapeDtypeStruct((N, H), jnp.float32),
        jax.ShapeDtypeStruct((N, NC_CLS), jnp.float32),
        jax.ShapeDtypeStruct((G, NC_CLS), jnp.float32),
    ],
)


def kernel(feature, edge_index, labels, W_emb, b_emb, Wg, bg, gam, bet,
           Wmg0, bmg0, Wmg1, bmg1, Wmg2, bmg2,
           Wmn0, bmn0, Wmn1, bmn1, Wmn2, bmn2):
    src = edge_index[0]
    dst = edge_index[1]
    srcr = src.reshape(NSUB, NCHUNK, CH)
    src_w = jnp.broadcast_to(
        srcr[None], (NCORES, NSUB, NCHUNK, CH)).reshape(NW, NCHUNK, CH)

    bases = (jnp.arange(NCORES, dtype=jnp.int32) * HR).reshape(NCORES, 1, 1, 1)

    dummy = HR + jnp.arange(CH, dtype=jnp.int32)

    def core_masked(idx):
        loc = idx.reshape(1, NSUB, NCHUNK, CH) - bases
        ok = (loc >= 0) & (loc < HR)
        return jnp.where(ok, loc, dummy).astype(jnp.int32).reshape(
            NW, NCHUNK, CH)

    srcm_w = core_masked(src)
    dstm_w = core_masked(dst)

    ones = jnp.ones((CH, DW), jnp.float32)
    zdeg = jnp.zeros((ZB, DW), jnp.float32)
    zrows = jnp.zeros((CH, H), jnp.float32)

    sc_degrees, sc_segsum = _sc_kernels()
    degp = sc_degrees(srcm_w, dstm_w, ones, zdeg)
    x, hpre, ns, nd = _tc_prep(feature, W_emb, b_emb, Wg[0], bg[0], degp)

    x3 = x_g = g = None
    for l in range(NL):
        p = sc_segsum(src_w, dstm_w, hpre, zrows)
        if l < NL - 1:
            x, hpre = _tc_layer(p, x, nd, gam[l], bet[l],
                                Wg[l + 1], bg[l + 1], ns)
        else:
            x3, x_g, g = _tc_final(p, x, nd, gam[l], bet[l],
                                   Wmg0, bmg0, Wmg1, bmg1, Wmg2, bmg2,
                                   Wmn0, bmn0, Wmn1, bmn1, Wmn2, bmn2)

    node_label = jnp.repeat(labels, NPG, axis=0)
    return x3, x_g, g, node_label
